# baseline (speedup 1.0000x reference)
import jax
import jax.numpy as jnp
from jax.experimental import pallas as pl

BATCH = 4
SEQ = 2048
D_MODEL = 1024
BS = 256


def _add_body(x_ref, pos_ref, o_ref):
    o_ref[...] = x_ref[...] + pos_ref[...][None, :, :]


def kernel(x, pos_table):
    return pl.pallas_call(
        _add_body,
        grid=(SEQ // BS, BATCH),
        in_specs=[
            pl.BlockSpec((1, BS, D_MODEL), lambda s, b: (b, s, 0)),
            pl.BlockSpec((BS, D_MODEL), lambda s, b: (s, 0)),
        ],
        out_specs=pl.BlockSpec((1, BS, D_MODEL), lambda s, b: (b, s, 0)),
        out_shape=jax.ShapeDtypeStruct((BATCH, SEQ, D_MODEL), jnp.float32),
    )(x, pos_table)
